# baseline (speedup 1.0000x reference)
import functools

import jax
import jax.numpy as jnp
from jax import lax
from jax.experimental import pallas as pl
from jax.experimental.pallas import tpu as pltpu
from jax.experimental.pallas import tpu_sc as plsc

N = 4096
D = 512
K = 2048

_NC, _NS = 2, 16
_NW = _NC * _NS
_BPW = K // _NW
_CH = 8
_NCHUNK = _BPW // _CH


def _gather_body(a_hbm, idx_hbm, out_hbm, idx_v, buf0, buf1, gsem, wsem0, wsem1):
    wid = lax.axis_index("s") * _NC + lax.axis_index("c")
    base = wid * _BPW
    pltpu.sync_copy(idx_hbm.at[wid], idx_v)
    bufs = (buf0, buf1)
    wsems = (wsem0, wsem1)
    pending = [None, None]
    for c in range(_NCHUNK):
        slot = c % 2
        if pending[slot] is not None:
            pending[slot].wait()
        pltpu.async_copy(a_hbm.at[idx_v.at[c]], bufs[slot], gsem).wait()
        pending[slot] = pltpu.async_copy(
            bufs[slot], out_hbm.at[pl.ds(base + c * _CH, _CH)], wsems[slot])
    pending[0].wait()
    pending[1].wait()


def _gather_rows(a, idx2d):
    mesh = plsc.VectorSubcoreMesh(core_axis_name="c", subcore_axis_name="s")
    return pl.kernel(
        _gather_body,
        mesh=mesh,
        out_type=jax.ShapeDtypeStruct((K, N), jnp.float32),
        scratch_types=[
            pltpu.VMEM((_NCHUNK, _CH), jnp.int32),
            pltpu.VMEM((_CH, N), jnp.float32),
            pltpu.VMEM((_CH, N), jnp.float32),
            pltpu.SemaphoreType.DMA,
            pltpu.SemaphoreType.DMA,
            pltpu.SemaphoreType.DMA,
        ],
    )(a, idx2d)


_BM = 256
_BK = 512
_GM = K // _BM
_GK = N // _BK


def _mm_body(ag_ref, x_ref, wt_ref, b_ref, val_ref, out_ref, acc_ref):
    j = pl.program_id(1)

    @pl.when(j == 0)
    def _():
        acc_ref[...] = jnp.zeros_like(acc_ref)

    acc_ref[...] += jnp.dot(ag_ref[...], x_ref[...],
                            preferred_element_type=jnp.float32)

    @pl.when(j == _GK - 1)
    def _():
        h = jnp.dot(acc_ref[...], wt_ref[...],
                    preferred_element_type=jnp.float32)
        out_ref[...] = (h + b_ref[...]) * val_ref[...]


def _matmul(ag, x, wt, b2d, val2d):
    return pl.pallas_call(
        _mm_body,
        grid=(_GM, _GK),
        in_specs=[
            pl.BlockSpec((_BM, _BK), lambda i, j: (i, j)),
            pl.BlockSpec((_BK, D), lambda i, j: (j, 0)),
            pl.BlockSpec((D, D), lambda i, j: (0, 0)),
            pl.BlockSpec((1, D), lambda i, j: (0, 0)),
            pl.BlockSpec((_BM, 1), lambda i, j: (i, 0)),
        ],
        out_specs=pl.BlockSpec((_BM, D), lambda i, j: (i, 0)),
        out_shape=jax.ShapeDtypeStruct((K, D), jnp.float32),
        scratch_shapes=[pltpu.VMEM((_BM, D), jnp.float32)],
        compiler_params=pltpu.CompilerParams(
            dimension_semantics=("parallel", "arbitrary")),
    )(ag, x, wt, b2d, val2d)


def kernel(A, X, idx, value, W, b):
    idx2d = idx.astype(jnp.int32).reshape(_NW, _NCHUNK, _CH)
    ag = _gather_rows(A, idx2d)
    return _matmul(ag, X, W.T, b.reshape(1, D), value.reshape(K, 1))


# speedup vs baseline: 1.6261x; 1.6261x over previous
import functools

import jax
import jax.numpy as jnp
from jax import lax
from jax.experimental import pallas as pl
from jax.experimental.pallas import tpu as pltpu
from jax.experimental.pallas import tpu_sc as plsc

N = 4096
D = 512
K = 2048

_NC, _NS = 2, 16
_NW = _NC * _NS
_BPW = K // _NW
_CH = 8
_NCHUNK = _BPW // _CH


def _gather_body(a_hbm, idx_hbm, out_hbm, idx_v, buf0, buf1, gsem, wsem0, wsem1):
    wid = lax.axis_index("s") * _NC + lax.axis_index("c")
    base = wid * _BPW
    pltpu.sync_copy(idx_hbm.at[wid], idx_v)
    bufs = (buf0, buf1)
    wsems = (wsem0, wsem1)
    pending = [None, None]
    for c in range(_NCHUNK):
        slot = c % 2
        if pending[slot] is not None:
            pending[slot].wait()
        pltpu.async_copy(a_hbm.at[idx_v.at[c]], bufs[slot], gsem).wait()
        pending[slot] = pltpu.async_copy(
            bufs[slot], out_hbm.at[pl.ds(base + c * _CH, _CH)], wsems[slot])
    pending[0].wait()
    pending[1].wait()


def _gather_rows(a, idx2d):
    mesh = plsc.VectorSubcoreMesh(core_axis_name="c", subcore_axis_name="s")
    return pl.kernel(
        _gather_body,
        mesh=mesh,
        out_type=jax.ShapeDtypeStruct((K, N), jnp.float32),
        scratch_types=[
            pltpu.VMEM((_NCHUNK, _CH), jnp.int32),
            pltpu.VMEM((_CH, N), jnp.float32),
            pltpu.VMEM((_CH, N), jnp.float32),
            pltpu.SemaphoreType.DMA,
            pltpu.SemaphoreType.DMA,
            pltpu.SemaphoreType.DMA,
        ],
    )(a, idx2d)


_BM = 256
_GM = K // _BM


def _mm_body(ag_ref, x_ref, wt_ref, b_ref, val_ref, out_ref):
    ag_bf = ag_ref[...].astype(jnp.bfloat16)
    acc = jnp.dot(ag_bf, x_ref[...], preferred_element_type=jnp.float32)
    h = jnp.dot(acc.astype(jnp.bfloat16), wt_ref[...],
                preferred_element_type=jnp.float32)
    out_ref[...] = (h + b_ref[...]) * val_ref[...]


def _matmul(ag, x_bf, wt_bf, b2d, val2d):
    return pl.pallas_call(
        _mm_body,
        grid=(_GM,),
        in_specs=[
            pl.BlockSpec((_BM, N), lambda i: (i, 0)),
            pl.BlockSpec((N, D), lambda i: (0, 0)),
            pl.BlockSpec((D, D), lambda i: (0, 0)),
            pl.BlockSpec((1, D), lambda i: (0, 0)),
            pl.BlockSpec((_BM, 1), lambda i: (i, 0)),
        ],
        out_specs=pl.BlockSpec((_BM, D), lambda i: (i, 0)),
        out_shape=jax.ShapeDtypeStruct((K, D), jnp.float32),
        compiler_params=pltpu.CompilerParams(
            dimension_semantics=("arbitrary",)),
    )(ag, x_bf, wt_bf, b2d, val2d)


def kernel(A, X, idx, value, W, b):
    idx2d = idx.astype(jnp.int32).reshape(_NW, _NCHUNK, _CH)
    ag = _gather_rows(A, idx2d)
    return _matmul(ag, X.astype(jnp.bfloat16), W.T.astype(jnp.bfloat16),
                   b.reshape(1, D), value.reshape(K, 1))
